# baseline (speedup 1.0000x reference)
import functools

import jax
import jax.numpy as jnp
from jax import lax
from jax.experimental import pallas as pl
from jax.experimental.pallas import tpu as pltpu
from jax.experimental.pallas import tpu_sc as plsc

NUM_NODES = 100000
MEM = 128
RAW = 128
TIME = 128
B = 16384
IN_DIM = 2 * MEM + RAW + TIME

NW = 32
BPW = B // NW
NCH = BPW // 128
TPW = 3136
NPAD = NW * TPW
NVEC = B // 16
SENT = 4095
VBITS = 20

_mesh = plsc.VectorSubcoreMesh(core_axis_name="c", subcore_axis_name="s")


@functools.partial(
    pl.kernel,
    mesh=_mesh,
    out_type=[
        jax.ShapeDtypeStruct((B, MEM), jnp.float32),
        jax.ShapeDtypeStruct((B, MEM), jnp.float32),
        jax.ShapeDtypeStruct((B, MEM), jnp.float32),
        jax.ShapeDtypeStruct((B,), jnp.int32),
    ],
    scratch_types=[
        pltpu.VMEM((BPW,), jnp.int32),
        pltpu.VMEM((BPW,), jnp.int32),
        pltpu.VMEM((BPW,), jnp.int32),
        pltpu.VMEM((BPW,), jnp.int32),
        pltpu.VMEM((BPW, MEM), jnp.float32),
        pltpu.SemaphoreType.DMA,
    ],
    compiler_params=pltpu.CompilerParams(needs_layout_passes=False),
)
def _sc_gather(mem_hbm, lu_hbm, nid_hbm, ds_hbm, dd_hbm,
               mn_out, ms_out, md_out, lun_out,
               nid_w, ds_w, dd_w, lu_w, rows, sem):
    wid = lax.axis_index("s") * 2 + lax.axis_index("c")
    base = wid * BPW

    pltpu.sync_copy(nid_hbm.at[pl.ds(base, BPW)], nid_w)
    pltpu.sync_copy(ds_hbm.at[pl.ds(base, BPW)], ds_w)
    pltpu.sync_copy(dd_hbm.at[pl.ds(base, BPW)], dd_w)

    cps = [pltpu.async_copy(lu_hbm.at[nid_w.at[pl.ds(c * 128, 128)]],
                            lu_w.at[pl.ds(c * 128, 128)], sem)
           for c in range(NCH)]
    for cp in cps:
        cp.wait()
    pltpu.sync_copy(lu_w, lun_out.at[pl.ds(base, BPW)])

    def gather_rows(idx_ref, out_ref):
        cs = [pltpu.async_copy(mem_hbm.at[idx_ref.at[pl.ds(c * 128, 128)]],
                               rows.at[pl.ds(c * 128, 128)], sem)
              for c in range(NCH)]
        for cp in cs:
            cp.wait()
        pltpu.sync_copy(rows, out_ref.at[pl.ds(base, BPW)])

    gather_rows(nid_w, mn_out)
    gather_rows(ds_w, ms_out)
    gather_rows(dd_w, md_out)


@functools.partial(
    pl.kernel,
    mesh=_mesh,
    out_type=[
        jax.ShapeDtypeStruct((B + 128,), jnp.int32),
    ],
    scratch_types=[
        pltpu.VMEM((B,), jnp.int32),
        pltpu.VMEM((B,), jnp.int32),
        pltpu.VMEM((B,), jnp.int32),
        pltpu.VMEM((TPW,), jnp.int32),
        pltpu.VMEM((B // 128, 128), jnp.int32),
        pltpu.VMEM((B // 128, 128), jnp.int32),
        pltpu.SemaphoreType.DMA,
    ],
    compiler_params=pltpu.CompilerParams(needs_layout_passes=False),
)
def _sc_scatmax(nid_hbm, ts_hbm, td_hbm, order_dep, nlu_out,
                nid_all, ts_all, td_all, table, posb, valb, sem):
    del order_dep
    wid = lax.axis_index("s") * 2 + lax.axis_index("c")

    pltpu.sync_copy(nid_hbm, nid_all)
    pltpu.sync_copy(ts_hbm, ts_all)
    pltpu.sync_copy(td_hbm, td_all)

    def zero_body(i, carry):
        table[pl.ds(i * 16, 16)] = jnp.zeros((16,), jnp.int32)
        return carry

    lax.fori_loop(0, TPW // 16, zero_body, 0)

    lo = wid * TPW
    lane = lax.iota(jnp.int32, 16)
    rotkey = (lane + 15) & 15

    def ev_body(i, carry):
        idx = nid_all[pl.ds(i * 16, 16)]
        tsv = ts_all[pl.ds(i * 16, 16)]
        tdv = td_all[pl.ds(i * 16, 16)]
        val = jnp.maximum(tsv, tdv)
        rel = idx - lo
        inr = (rel >= 0) & (rel < TPW)
        keyp = jnp.where(inr, rel, SENT)
        comp = (keyp.astype(jnp.uint32) << VBITS) | val.astype(jnp.uint32)
        compi = comp.astype(jnp.int32)
        _, s32 = plsc.sort_key_val(compi, compi)
        s = s32.astype(jnp.uint32)
        k2 = (s >> VBITS).astype(jnp.int32)
        v2 = (s & ((1 << VBITS) - 1)).astype(jnp.int32)
        _, rot = plsc.sort_key_val(rotkey, s)
        nk = (rot >> VBITS).astype(jnp.int32)
        last = (k2 != nk) | (lane == 15)
        wm = last & (k2 != SENT)
        skc = jnp.minimum(k2, TPW - 1)
        cur = plsc.load_gather(table, [skc])
        newv = jnp.maximum(cur, v2)
        plsc.store_scatter(table, [skc], newv, mask=wm)
        return carry

    lax.fori_loop(0, NVEC, ev_body, 0)

    def ans_body(i, cntv):
        idx = nid_all[pl.ds(i * 16, 16)]
        rel = idx - lo
        inr = (rel >= 0) & (rel < TPW)
        relc = jnp.minimum(jnp.maximum(rel, 0), TPW - 1)
        vals = plsc.load_gather(table, [relc])
        rank_incl = plsc.cumsum(inr.astype(jnp.int32))
        slot = jnp.maximum(cntv + rank_incl - 1, 0)
        row = jnp.minimum(slot >> 7, B // 128 - 1)
        col = slot & 127
        pos = i * 16 + lane
        plsc.store_scatter(posb, [row, col], pos, mask=inr)
        plsc.store_scatter(valb, [row, col], vals, mask=inr)
        lane15 = jnp.full((16,), 15, jnp.int32)
        tot = rank_incl.at[lane15].get(mode="promise_in_bounds")
        return cntv + tot

    cntv = lax.fori_loop(0, NVEC, ans_body, jnp.zeros((16,), jnp.int32))
    cnt = jnp.max(cntv)

    ndma = (cnt + 127) >> 7
    limit = ndma << 7
    sinkpos = jnp.full((16,), B, jnp.int32)
    for j in range(8):
        slot = cnt + j * 16 + lane
        wm = slot < limit
        rowp = jnp.minimum(slot >> 7, B // 128 - 1)
        colp = slot & 127
        plsc.store_scatter(posb, [rowp, colp], sinkpos, mask=wm)

    def dma_body(r, carry):
        pltpu.async_copy(valb.at[r], nlu_out.at[posb.at[r]], sem).wait()
        return carry

    lax.fori_loop(0, ndma, dma_body, 0)


BLK = 1024


def _fast_cos(x):
    n = jnp.round(x * 0.15915494)
    nh = jnp.floor(n * 0.0009765625)
    nl = n - nh * 1024.0
    r = x - nh * 6432.0
    r = r - nl * 6.28125
    r = r - n * 0.0019353072
    u = r * r
    p = -2.0301664e-07
    p = p * u + 2.3758734e-05
    p = p * u - 0.0013816874
    p = p * u + 0.041643132
    p = p * u - 0.49996909
    p = p * u + 0.99999028
    return p


def _tc_body(mn, ms, md, rs, rd, ts, td, lu, tw, tb, wih, whh, bih, bhh, out):
    h = mn[...]
    dst = 0.5 * (ms[...] + md[...])
    raw = 0.5 * (rs[...] + rd[...])
    twv = tw[...]
    tbv = tb[...]
    tsc = jnp.transpose((ts[...] - lu[...]).reshape(1, BLK))
    tdc = jnp.transpose((td[...] - lu[...]).reshape(1, BLK))
    trel_s = tsc.astype(jnp.float32)
    trel_d = tdc.astype(jnp.float32)
    enc = 0.5 * (_fast_cos(trel_s * twv + tbv) + _fast_cos(trel_d * twv + tbv))
    aggr = jnp.concatenate([h, dst, raw, enc], axis=1)
    gi = lax.dot_general(aggr, wih[...], (((1,), (1,)), ((), ())),
                         preferred_element_type=jnp.float32) + bih[...]
    gh = lax.dot_general(h, whh[...], (((1,), (1,)), ((), ())),
                         preferred_element_type=jnp.float32) + bhh[...]
    r = jax.nn.sigmoid(gi[:, :MEM] + gh[:, :MEM])
    z = jax.nn.sigmoid(gi[:, MEM:2 * MEM] + gh[:, MEM:2 * MEM])
    n = jnp.tanh(gi[:, 2 * MEM:] + r * gh[:, 2 * MEM:])
    out[...] = (1.0 - z) * n + z * h


def _tc_dense(mn, ms, md, rs, rd, ts2, td2, lu2, tw2, tb2, wih, whh, bih2, bhh2):
    bs_feat = pl.BlockSpec((BLK, MEM), lambda i: (i, 0))
    bs_row = pl.BlockSpec((1, 1, BLK), lambda i: (i, 0, 0))

    def const(shape):
        return pl.BlockSpec(shape, lambda i: (0, 0))

    return pl.pallas_call(
        _tc_body,
        grid=(B // BLK,),
        in_specs=[bs_feat] * 5 + [bs_row] * 3 + [
            const((1, TIME)), const((1, TIME)),
            const((3 * MEM, IN_DIM)), const((3 * MEM, MEM)),
            const((1, 3 * MEM)), const((1, 3 * MEM)),
        ],
        out_specs=bs_feat,
        out_shape=jax.ShapeDtypeStruct((B, MEM), jnp.float32),
    )(mn, ms, md, rs, rd, ts2, td2, lu2, tw2, tb2, wih, whh, bih2, bhh2)


def kernel(n_id, dst_s, dst_d, t_s, t_d, raw_msg_s, raw_msg_d, memory,
           last_update, time_w, time_b, w_ih, w_hh, b_ih, b_hh):
    mn, ms, md, lun = _sc_gather(memory, last_update, n_id, dst_s, dst_d)
    (nlu_full,) = _sc_scatmax(n_id, t_s, t_d, mn)
    nlu = nlu_full[:B]
    new_memory = _tc_dense(
        mn, ms, md, raw_msg_s, raw_msg_d,
        t_s.reshape(B // BLK, 1, BLK), t_d.reshape(B // BLK, 1, BLK),
        lun.reshape(B // BLK, 1, BLK),
        time_w.reshape(1, TIME), time_b.reshape(1, TIME),
        w_ih, w_hh, b_ih.reshape(1, 3 * MEM), b_hh.reshape(1, 3 * MEM))
    return new_memory, nlu


# speedup vs baseline: 3.8134x; 3.8134x over previous
import functools

import jax
import jax.numpy as jnp
from jax import lax
from jax.experimental import pallas as pl
from jax.experimental.pallas import tpu as pltpu
from jax.experimental.pallas import tpu_sc as plsc

NUM_NODES = 100000
MEM = 128
RAW = 128
TIME = 128
B = 16384
IN_DIM = 2 * MEM + RAW + TIME

NW = 32
BPW = B // NW
NCH = BPW // 128
TPW = 3136
NPAD = NW * TPW
NVEC = B // 16
SENT = 4095
VBITS = 20

_mesh = plsc.VectorSubcoreMesh(core_axis_name="c", subcore_axis_name="s")


@functools.partial(
    pl.kernel,
    mesh=_mesh,
    out_type=[
        jax.ShapeDtypeStruct((B, MEM), jnp.float32),
        jax.ShapeDtypeStruct((B, MEM), jnp.float32),
        jax.ShapeDtypeStruct((B, MEM), jnp.float32),
        jax.ShapeDtypeStruct((B,), jnp.int32),
    ],
    scratch_types=[
        pltpu.VMEM((BPW,), jnp.int32),
        pltpu.VMEM((BPW,), jnp.int32),
        pltpu.VMEM((BPW,), jnp.int32),
        pltpu.VMEM((BPW,), jnp.int32),
        pltpu.VMEM((BPW, MEM), jnp.float32),
        pltpu.SemaphoreType.DMA,
    ],
    compiler_params=pltpu.CompilerParams(needs_layout_passes=False),
)
def _sc_gather(mem_hbm, lu_hbm, nid_hbm, ds_hbm, dd_hbm,
               mn_out, ms_out, md_out, lun_out,
               nid_w, ds_w, dd_w, lu_w, rows, sem):
    wid = lax.axis_index("s") * 2 + lax.axis_index("c")
    base = wid * BPW

    pltpu.sync_copy(nid_hbm.at[pl.ds(base, BPW)], nid_w)
    pltpu.sync_copy(ds_hbm.at[pl.ds(base, BPW)], ds_w)
    pltpu.sync_copy(dd_hbm.at[pl.ds(base, BPW)], dd_w)

    cps = [pltpu.async_copy(lu_hbm.at[nid_w.at[pl.ds(c * 128, 128)]],
                            lu_w.at[pl.ds(c * 128, 128)], sem)
           for c in range(NCH)]
    for cp in cps:
        cp.wait()
    pltpu.sync_copy(lu_w, lun_out.at[pl.ds(base, BPW)])

    def gather_rows(idx_ref, out_ref):
        cs = [pltpu.async_copy(mem_hbm.at[idx_ref.at[pl.ds(c * 128, 128)]],
                               rows.at[pl.ds(c * 128, 128)], sem)
              for c in range(NCH)]
        for cp in cs:
            cp.wait()
        pltpu.sync_copy(rows, out_ref.at[pl.ds(base, BPW)])

    gather_rows(nid_w, mn_out)
    gather_rows(ds_w, ms_out)
    gather_rows(dd_w, md_out)


@functools.partial(
    pl.kernel,
    mesh=_mesh,
    out_type=[
        jax.ShapeDtypeStruct((NPAD,), jnp.int32),
    ],
    scratch_types=[
        pltpu.VMEM((B,), jnp.int32),
        pltpu.VMEM((B,), jnp.int32),
        pltpu.VMEM((B,), jnp.int32),
        pltpu.VMEM((TPW,), jnp.int32),
    ],
    compiler_params=pltpu.CompilerParams(needs_layout_passes=False),
)
def _sc_scatmax(nid_hbm, ts_hbm, td_hbm, order_dep, tab_out,
                nid_all, ts_all, td_all, table):
    del order_dep
    wid = lax.axis_index("s") * 2 + lax.axis_index("c")

    pltpu.sync_copy(nid_hbm, nid_all)
    pltpu.sync_copy(ts_hbm, ts_all)
    pltpu.sync_copy(td_hbm, td_all)

    def zero_body(i, carry):
        table[pl.ds(i * 16, 16)] = jnp.zeros((16,), jnp.int32)
        return carry

    lax.fori_loop(0, TPW // 16, zero_body, 0)

    lo = wid * TPW
    lane = lax.iota(jnp.int32, 16)
    rotkey = (lane + 15) & 15

    def ev_body(i, carry):
        idx = nid_all[pl.ds(i * 16, 16)]
        tsv = ts_all[pl.ds(i * 16, 16)]
        tdv = td_all[pl.ds(i * 16, 16)]
        val = jnp.maximum(tsv, tdv)
        rel = idx - lo
        inr = (rel >= 0) & (rel < TPW)
        keyp = jnp.where(inr, rel, SENT)
        comp = (keyp.astype(jnp.uint32) << VBITS) | val.astype(jnp.uint32)
        compi = comp.astype(jnp.int32)
        _, s32 = plsc.sort_key_val(compi, compi)
        s = s32.astype(jnp.uint32)
        k2 = (s >> VBITS).astype(jnp.int32)
        v2 = (s & ((1 << VBITS) - 1)).astype(jnp.int32)
        _, rot = plsc.sort_key_val(rotkey, s)
        nk = (rot >> VBITS).astype(jnp.int32)
        last = (k2 != nk) | (lane == 15)
        wm = last & (k2 != SENT)
        skc = jnp.minimum(k2, TPW - 1)
        cur = plsc.load_gather(table, [skc])
        newv = jnp.maximum(cur, v2)
        plsc.store_scatter(table, [skc], newv, mask=wm)
        return carry

    lax.fori_loop(0, NVEC, ev_body, 0)

    pltpu.sync_copy(table, tab_out.at[pl.ds(lo, TPW)])


@functools.partial(
    pl.kernel,
    mesh=_mesh,
    out_type=[jax.ShapeDtypeStruct((B,), jnp.int32)],
    scratch_types=[
        pltpu.VMEM((BPW,), jnp.int32),
        pltpu.VMEM((BPW,), jnp.int32),
        pltpu.SemaphoreType.DMA,
    ],
)
def _sc_lookup(tab_hbm, nid_hbm, out_hbm, idx_w, res_w, sem):
    wid = lax.axis_index("s") * 2 + lax.axis_index("c")
    base = wid * BPW
    pltpu.sync_copy(nid_hbm.at[pl.ds(base, BPW)], idx_w)
    cps = [pltpu.async_copy(tab_hbm.at[idx_w.at[pl.ds(c * 128, 128)]],
                            res_w.at[pl.ds(c * 128, 128)], sem)
           for c in range(NCH)]
    for cp in cps:
        cp.wait()
    pltpu.sync_copy(res_w, out_hbm.at[pl.ds(base, BPW)])


BLK = 1024


def _fast_cos(x):
    n = jnp.round(x * 0.15915494)
    nh = jnp.floor(n * 0.0009765625)
    nl = n - nh * 1024.0
    r = x - nh * 6432.0
    r = r - nl * 6.28125
    r = r - n * 0.0019353072
    u = r * r
    p = -2.0301664e-07
    p = p * u + 2.3758734e-05
    p = p * u - 0.0013816874
    p = p * u + 0.041643132
    p = p * u - 0.49996909
    p = p * u + 0.99999028
    return p


def _tc_body(mn, ms, md, rs, rd, ts, td, lu, tw, tb, wih, whh, bih, bhh, out):
    h = mn[...]
    dst = 0.5 * (ms[...] + md[...])
    raw = 0.5 * (rs[...] + rd[...])
    twv = tw[...]
    tbv = tb[...]
    tsc = jnp.transpose((ts[...] - lu[...]).reshape(1, BLK))
    tdc = jnp.transpose((td[...] - lu[...]).reshape(1, BLK))
    trel_s = tsc.astype(jnp.float32)
    trel_d = tdc.astype(jnp.float32)
    enc = 0.5 * (_fast_cos(trel_s * twv + tbv) + _fast_cos(trel_d * twv + tbv))
    aggr = jnp.concatenate([h, dst, raw, enc], axis=1)
    gi = lax.dot_general(aggr, wih[...], (((1,), (1,)), ((), ())),
                         preferred_element_type=jnp.float32) + bih[...]
    gh = lax.dot_general(h, whh[...], (((1,), (1,)), ((), ())),
                         preferred_element_type=jnp.float32) + bhh[...]
    r = jax.nn.sigmoid(gi[:, :MEM] + gh[:, :MEM])
    z = jax.nn.sigmoid(gi[:, MEM:2 * MEM] + gh[:, MEM:2 * MEM])
    n = jnp.tanh(gi[:, 2 * MEM:] + r * gh[:, 2 * MEM:])
    out[...] = (1.0 - z) * n + z * h


def _tc_dense(mn, ms, md, rs, rd, ts2, td2, lu2, tw2, tb2, wih, whh, bih2, bhh2):
    bs_feat = pl.BlockSpec((BLK, MEM), lambda i: (i, 0))
    bs_row = pl.BlockSpec((1, 1, BLK), lambda i: (i, 0, 0))

    def const(shape):
        return pl.BlockSpec(shape, lambda i: (0, 0))

    return pl.pallas_call(
        _tc_body,
        grid=(B // BLK,),
        in_specs=[bs_feat] * 5 + [bs_row] * 3 + [
            const((1, TIME)), const((1, TIME)),
            const((3 * MEM, IN_DIM)), const((3 * MEM, MEM)),
            const((1, 3 * MEM)), const((1, 3 * MEM)),
        ],
        out_specs=bs_feat,
        out_shape=jax.ShapeDtypeStruct((B, MEM), jnp.float32),
    )(mn, ms, md, rs, rd, ts2, td2, lu2, tw2, tb2, wih, whh, bih2, bhh2)


def kernel(n_id, dst_s, dst_d, t_s, t_d, raw_msg_s, raw_msg_d, memory,
           last_update, time_w, time_b, w_ih, w_hh, b_ih, b_hh):
    mn, ms, md, lun = _sc_gather(memory, last_update, n_id, dst_s, dst_d)
    (tab,) = _sc_scatmax(n_id, t_s, t_d, mn)
    (nlu,) = _sc_lookup(tab, n_id)
    new_memory = _tc_dense(
        mn, ms, md, raw_msg_s, raw_msg_d,
        t_s.reshape(B // BLK, 1, BLK), t_d.reshape(B // BLK, 1, BLK),
        lun.reshape(B // BLK, 1, BLK),
        time_w.reshape(1, TIME), time_b.reshape(1, TIME),
        w_ih, w_hh, b_ih.reshape(1, 3 * MEM), b_hh.reshape(1, 3 * MEM))
    return new_memory, nlu
